# Initial kernel scaffold; baseline (speedup 1.0000x reference)
import jax
import jax.numpy as jnp
from jax.experimental import pallas as pl


def kernel(x, table, W1, b1, W2, b2):
    raise NotImplementedError("write your pallas kernel here")


# speedup vs baseline: 4.5112x; 4.5112x over previous
import functools

import jax
import jax.numpy as jnp
from jax import lax
from jax.experimental import pallas as pl
from jax.experimental.pallas import tpu as pltpu
from jax.experimental.pallas import tpu_sc as plsc

VOCAB = 100000
EMBED = 64
BATCH = 4096
HIST = 50
TOT = BATCH * HIST
CHUNK = 128
NW = 32
CPW = TOT // (CHUNK * NW)


def _sc_gather(x3d, table):
    mesh = plsc.VectorSubcoreMesh(core_axis_name="c", subcore_axis_name="s")

    @functools.partial(
        pl.kernel,
        mesh=mesh,
        compiler_params=pltpu.CompilerParams(use_tc_tiling_on_sc=False),
        out_type=jax.ShapeDtypeStruct((TOT, EMBED), jnp.float32),
        scratch_types=[
            pltpu.VMEM((CPW, CHUNK), jnp.int32),
            pltpu.VMEM((CHUNK, EMBED), jnp.float32),
            pltpu.SemaphoreType.DMA,
        ],
    )
    def k(idx_hbm, table_hbm, out_hbm, idx_v, rows_v, sem):
        wid = lax.axis_index("s") * 2 + lax.axis_index("c")
        row0 = wid * CPW
        pltpu.sync_copy(idx_hbm.at[wid], idx_v)

        def body(j, carry):
            pltpu.async_copy(table_hbm.at[idx_v.at[j]], rows_v, sem).wait()
            pltpu.sync_copy(rows_v, out_hbm.at[pl.ds((row0 + j) * CHUNK, CHUNK)])
            return carry

        lax.fori_loop(0, CPW, body, 0)

    return k(x3d, table)


def _mlp_block(flat_ref, w1_ref, b1_ref, w2_ref, b2_ref, out_ref):
    h = jnp.dot(flat_ref[...], w1_ref[...], preferred_element_type=jnp.float32)
    h = jnp.maximum(h + b1_ref[...], 0.0)
    o = jnp.dot(h, w2_ref[...], preferred_element_type=jnp.float32)
    out_ref[...] = jnp.maximum(o + b2_ref[...], 0.0)


def _tc_mlp(flat, W1, b1, W2, b2):
    BB = 512
    grid = (BATCH // BB,)
    return pl.pallas_call(
        _mlp_block,
        grid=grid,
        in_specs=[
            pl.BlockSpec((BB, HIST * EMBED), lambda i: (i, 0)),
            pl.BlockSpec((HIST * EMBED, 1024), lambda i: (0, 0)),
            pl.BlockSpec((1, 1024), lambda i: (0, 0)),
            pl.BlockSpec((1024, 512), lambda i: (0, 0)),
            pl.BlockSpec((1, 512), lambda i: (0, 0)),
        ],
        out_specs=pl.BlockSpec((BB, 512), lambda i: (i, 0)),
        out_shape=jax.ShapeDtypeStruct((BATCH, 512), jnp.float32),
    )(flat, W1, b1.reshape(1, -1), W2, b2.reshape(1, -1))


def kernel(x, table, W1, b1, W2, b2):
    x3d = x.reshape(NW, CPW, CHUNK)
    rows = _sc_gather(x3d, table)
    flat = rows.reshape(BATCH, HIST * EMBED)
    return _tc_mlp(flat, W1, b1, W2, b2)


# speedup vs baseline: 5.0729x; 1.1245x over previous
import functools

import jax
import jax.numpy as jnp
from jax import lax
from jax.experimental import pallas as pl
from jax.experimental.pallas import tpu as pltpu
from jax.experimental.pallas import tpu_sc as plsc

VOCAB = 100000
EMBED = 64
BATCH = 4096
HIST = 50
TOT = BATCH * HIST
CHUNK = 128
NW = 32
CPW = TOT // (CHUNK * NW)


def _sc_gather(x3d, table):
    mesh = plsc.VectorSubcoreMesh(core_axis_name="c", subcore_axis_name="s")

    @functools.partial(
        pl.kernel,
        mesh=mesh,
        compiler_params=pltpu.CompilerParams(use_tc_tiling_on_sc=False),
        out_type=jax.ShapeDtypeStruct((TOT, EMBED), jnp.float32),
        scratch_types=[
            pltpu.VMEM((CPW, CHUNK), jnp.int32),
            pltpu.VMEM((CHUNK, EMBED), jnp.float32),
            pltpu.VMEM((CHUNK, EMBED), jnp.float32),
            pltpu.SemaphoreType.DMA,
            pltpu.SemaphoreType.DMA,
            pltpu.SemaphoreType.DMA,
            pltpu.SemaphoreType.DMA,
        ],
    )
    def k(idx_hbm, table_hbm, out_hbm, idx_v, buf_a, buf_b, ga, gb, oa, ob):
        wid = lax.axis_index("s") * 2 + lax.axis_index("c")
        row0 = wid * CPW
        pltpu.sync_copy(idx_hbm.at[wid], idx_v)

        def start_gather(j, buf, sem):
            pltpu.async_copy(table_hbm.at[idx_v.at[j]], buf, sem)

        def start_out(j, buf, sem):
            pltpu.async_copy(buf, out_hbm.at[pl.ds((row0 + j) * CHUNK, CHUNK)], sem)

        def wait_gather(buf, sem):
            pltpu.make_async_copy(out_hbm.at[pl.ds(0, CHUNK)], buf, sem).wait()

        def wait_out(buf, sem):
            pltpu.make_async_copy(buf, out_hbm.at[pl.ds(0, CHUNK)], sem).wait()

        start_gather(0, buf_a, ga)

        def body(g, carry):
            j0 = 2 * g

            @pl.when(g > 0)
            def _():
                wait_out(buf_b, ob)

            start_gather(j0 + 1, buf_b, gb)
            wait_gather(buf_a, ga)
            start_out(j0, buf_a, oa)
            wait_out(buf_a, oa)

            @pl.when(j0 + 2 < CPW)
            def _():
                start_gather(j0 + 2, buf_a, ga)

            wait_gather(buf_b, gb)
            start_out(j0 + 1, buf_b, ob)
            return carry

        lax.fori_loop(0, CPW // 2, body, 0)
        wait_out(buf_b, ob)

    return k(x3d, table)


def _mlp_block(flat_ref, w1_ref, b1_ref, w2_ref, b2_ref, out_ref):
    h = jnp.dot(flat_ref[...], w1_ref[...], preferred_element_type=jnp.float32)
    h = jnp.maximum(h + b1_ref[...], 0.0)
    o = jnp.dot(h, w2_ref[...], preferred_element_type=jnp.float32)
    out_ref[...] = jnp.maximum(o + b2_ref[...], 0.0)


def _tc_mlp(flat, W1, b1, W2, b2):
    BB = 512
    grid = (BATCH // BB,)
    return pl.pallas_call(
        _mlp_block,
        grid=grid,
        in_specs=[
            pl.BlockSpec((BB, HIST * EMBED), lambda i: (i, 0)),
            pl.BlockSpec((HIST * EMBED, 1024), lambda i: (0, 0)),
            pl.BlockSpec((1, 1024), lambda i: (0, 0)),
            pl.BlockSpec((1024, 512), lambda i: (0, 0)),
            pl.BlockSpec((1, 512), lambda i: (0, 0)),
        ],
        out_specs=pl.BlockSpec((BB, 512), lambda i: (i, 0)),
        out_shape=jax.ShapeDtypeStruct((BATCH, 512), jnp.float32),
    )(flat, W1, b1.reshape(1, -1), W2, b2.reshape(1, -1))


def kernel(x, table, W1, b1, W2, b2):
    x3d = x.reshape(NW, CPW, CHUNK)
    rows = _sc_gather(x3d, table)
    flat = rows.reshape(BATCH, HIST * EMBED)
    return _tc_mlp(flat, W1, b1, W2, b2)
